# baseline (speedup 1.0000x reference)
import jax
import jax.numpy as jnp
from jax import lax
from jax.experimental import pallas as pl
from jax.experimental.pallas import tpu as pltpu
from jax.experimental.pallas import tpu_sc as plsc

_N = 5000
_PAD = 5120
_T = 0.5
_MAX_ROUNDS = 100
_NEG = -1e30
_NSUB = 16
_L = 16
_SLICE = _PAD // _NSUB
_CHUNKS = _SLICE // _L
_OROWS = _PAD // _NSUB
_OW = 5
_LAST = _NSUB - 1
_LASTW = (_N - _LAST * _OROWS) * _OW


def _sc_body(bh, sh, cnth, outh, bv, sv_, cntv, alive, outv, stage, candv,
             cands):
    sid = lax.axis_index("s")
    base = sid * _SLICE
    l16 = lax.iota(jnp.int32, 16)
    l16x4 = l16 * 4
    zeros16 = jnp.zeros((16,), jnp.float32)
    obase_v = jnp.full((16,), sid * _OROWS, jnp.int32)

    pltpu.sync_copy(bh.at[pl.ds(base * 4, _SLICE * 4)], bv)
    pltpu.sync_copy(sh.at[pl.ds(base, _SLICE)], sv_)
    pltpu.sync_copy(cnth, cntv)
    cntvec = cntv[...]

    for j in range(_CHUNKS):
        iv = base + j * _L + l16
        alive[pl.ds(j * _L, _L)] = jnp.where(iv < _N, 1.0, 0.0)
    for r in range(_OROWS * _OW // _L):
        outv[pl.ds(r * _L, _L)] = zeros16

    def round_body(t, carry):
        ipv, x1w, y1w, x2w, y2w, sw, aw = carry
        vwv = ipv < _N
        mv = jnp.full((16,), _NEG, jnp.float32)
        mivf = jnp.full((16,), 1e9, jnp.float32)
        prv = zeros16
        for j in range(_CHUNKS):
            sl = pl.ds(j * _L, _L)
            iv = base + j * _L + l16
            bix = j * 64 + l16x4
            av = alive[sl]
            xv1 = plsc.load_gather(bv, [bix])
            yv1 = plsc.load_gather(bv, [bix + 1])
            xv2 = plsc.load_gather(bv, [bix + 2])
            yv2 = plsc.load_gather(bv, [bix + 3])
            scv = sv_[sl]
            w = jnp.maximum(jnp.minimum(xv2, x2w) - jnp.maximum(xv1, x1w), 0.0)
            h = jnp.maximum(jnp.minimum(yv2, y2w) - jnp.maximum(yv1, y1w), 0.0)
            inter = w * h
            areav = (xv2 - xv1) * (yv2 - yv1)
            denom = jnp.maximum(areav + aw - inter, 1e-9)
            anew = jnp.where(
                (iv != ipv) & ~((inter > _T * denom) & vwv) & (av > 0.5),
                1.0, 0.0)
            alive[sl] = anew
            hi = (scv > sw) | ((scv == sw) & (iv < ipv))
            prv = prv + jnp.where(hi, 1.0, 0.0)
            cand = jnp.where(anew > 0.5, scv, _NEG)
            upd = cand > mv
            mivf = jnp.where(upd, iv.astype(jnp.float32), mivf)
            mv = jnp.maximum(mv, cand)
        mloc = jnp.max(mv)
        ilocf = jnp.min(jnp.where(mv == mloc, mivf, jnp.float32(1e9)))
        prf = jnp.sum(prv)
        li4 = jnp.clip(jnp.full((16,), ilocf.astype(jnp.int32)) - base,
                       0, _SLICE - 1) * 4
        xc1 = plsc.load_gather(bv, [li4])
        yc1 = plsc.load_gather(bv, [li4 + 1])
        xc2 = plsc.load_gather(bv, [li4 + 2])
        yc2 = plsc.load_gather(bv, [li4 + 3])
        row = jnp.where(l16 == 0, mloc,
                        jnp.where(l16 == 1, ilocf,
                                  jnp.where(l16 == 2, prf,
                                            jnp.where(l16 == 3, xc1,
                                                      jnp.where(l16 == 4, yc1,
                                                                jnp.where(l16 == 5, xc2,
                                                                          jnp.where(l16 == 6, yc2,
                                                                                    0.0)))))))
        stage[...] = row
        p = t & 1
        pltpu.sync_copy(stage, cands.at[p, sid])
        plsc.subcore_barrier()
        pltpu.sync_copy(cands.at[p], candv)
        zi = jnp.zeros((16,), jnp.int32)
        sc16 = plsc.load_gather(candv, [l16, zi])
        ixf16 = plsc.load_gather(candv, [l16, zi + 1])
        pr16 = plsc.load_gather(candv, [l16, zi + 2])
        mg = jnp.max(sc16)
        winm = sc16 == mg
        iwf = jnp.min(jnp.where(winm, ixf16, jnp.float32(1e9)))
        iw = jnp.full((16,), iwf.astype(jnp.int32))
        rank = jnp.full((16,), jnp.sum(pr16).astype(jnp.int32))
        l16f = l16.astype(jnp.float32)
        wlf = jnp.min(jnp.where(winm & (ixf16 == iwf), l16f,
                                jnp.float32(1e9)))
        wlv = jnp.full((16,), wlf.astype(jnp.int32))
        tv = jnp.full((16,), t, jnp.int32)
        rec = vwv & (tv - 1 < cntvec) & (rank >= obase_v) & \
              (rank < obase_v + _OROWS)
        rrow = jnp.where(l16 == 0, x1w,
                         jnp.where(l16 == 1, y1w,
                                   jnp.where(l16 == 2, x2w,
                                             jnp.where(l16 == 3, y2w,
                                                       jnp.where(l16 == 4, sw,
                                                                 0.0)))))
        ridx = jnp.where(rec, (rank - obase_v) * _OW, 0) + l16
        plsc.store_scatter(outv, [ridx], rrow, mask=rec & (l16 < _OW))
        x1n = plsc.load_gather(candv, [wlv, zi + 3])
        y1n = plsc.load_gather(candv, [wlv, zi + 4])
        x2n = plsc.load_gather(candv, [wlv, zi + 5])
        y2n = plsc.load_gather(candv, [wlv, zi + 6])
        sn = jnp.full((16,), mg)
        an = (x2n - x1n) * (y2n - y1n)
        return iw, x1n, y1n, x2n, y2n, sn, an

    big = jnp.full((16,), 3e38, jnp.float32)
    bigi = jnp.full((16,), 2**30, jnp.int32)
    carry = (bigi, zeros16, zeros16, zeros16, zeros16, big, zeros16)
    lax.fori_loop(0, _MAX_ROUNDS + 1, round_body, carry)

    @pl.when(sid < _LAST)
    def _():
        pltpu.sync_copy(outv,
                        outh.at[pl.ds(sid * _OROWS * _OW, _OROWS * _OW)])

    @pl.when(sid == _LAST)
    def _():
        pltpu.sync_copy(outv.at[pl.ds(0, _LASTW)],
                        outh.at[pl.ds(_LAST * _OROWS * _OW, _LASTW)])


@jax.jit
def _nms_sc(boxes, scores, counts):
    bflat = jnp.pad(boxes.reshape(-1), (0, (_PAD - _N) * 4))
    s = jnp.pad(scores, (0, _PAD - _N))
    cnt = jnp.full((16,), jnp.asarray(counts, jnp.int32))
    mesh = plsc.VectorSubcoreMesh(core_axis_name="c", subcore_axis_name="s",
                                  num_cores=1)
    f = pl.kernel(
        _sc_body,
        out_type=jax.ShapeDtypeStruct((_N * _OW,), jnp.float32),
        mesh=mesh,
        compiler_params=pltpu.CompilerParams(needs_layout_passes=False,
                                             use_tc_tiling_on_sc=False),
        scratch_types=[
            pltpu.VMEM((_SLICE * 4,), jnp.float32),
            pltpu.VMEM((_SLICE,), jnp.float32),
            pltpu.VMEM((16,), jnp.int32),
            pltpu.VMEM((_SLICE,), jnp.float32),
            pltpu.VMEM((_OROWS * _OW,), jnp.float32),
            pltpu.VMEM((16,), jnp.float32),
            pltpu.VMEM((16, 16), jnp.float32),
            pltpu.VMEM_SHARED((2, 16, 16), jnp.float32),
        ],
    )
    out = f(bflat, s, cnt)
    return out.reshape(_N, _OW)


def kernel(boxes, scores, counts):
    return _nms_sc(boxes, scores, counts)


# speedup vs baseline: 1.3028x; 1.3028x over previous
import jax
import jax.numpy as jnp
from jax import lax
from jax.experimental import pallas as pl
from jax.experimental.pallas import tpu as pltpu
from jax.experimental.pallas import tpu_sc as plsc

_N = 5000
_PAD = 5120
_T = 0.5
_MAX_ROUNDS = 100
_NEG = -1e30
_NSUB = 16
_L = 16
_SLICE = _PAD // _NSUB
_CHUNKS = _SLICE // _L
_OROWS = _PAD // _NSUB
_OW = 5
_LAST = _NSUB - 1
_LASTW = (_N - _LAST * _OROWS) * _OW


def _sc_body(bh, sh, cnth, outh, bv, x1v, y1v, x2v, y2v, sv_, cntv, alive,
             outv, stage, candv, cands):
    sid = lax.axis_index("s")
    base = sid * _SLICE
    l16 = lax.iota(jnp.int32, 16)
    l16x4 = l16 * 4
    zeros16 = jnp.zeros((16,), jnp.float32)
    obase_v = jnp.full((16,), sid * _OROWS, jnp.int32)

    pltpu.sync_copy(bh.at[pl.ds(base * 4, _SLICE * 4)], bv)
    pltpu.sync_copy(sh.at[pl.ds(base, _SLICE)], sv_)
    pltpu.sync_copy(cnth, cntv)
    cntvec = cntv[...]

    for j in range(_CHUNKS):
        sl = pl.ds(j * _L, _L)
        bix = j * 64 + l16x4
        x1v[sl] = plsc.load_gather(bv, [bix])
        y1v[sl] = plsc.load_gather(bv, [bix + 1])
        x2v[sl] = plsc.load_gather(bv, [bix + 2])
        y2v[sl] = plsc.load_gather(bv, [bix + 3])

    for j in range(_CHUNKS):
        iv = base + j * _L + l16
        alive[pl.ds(j * _L, _L)] = jnp.where(iv < _N, 1.0, 0.0)
    for r in range(_OROWS * _OW // _L):
        outv[pl.ds(r * _L, _L)] = zeros16

    def round_body(t, carry):
        ipv, x1w, y1w, x2w, y2w, sw, aw = carry
        vwv = ipv < _N
        mv = jnp.full((16,), _NEG, jnp.float32)
        mivf = jnp.full((16,), 1e9, jnp.float32)
        prv = zeros16
        for j in range(_CHUNKS):
            sl = pl.ds(j * _L, _L)
            iv = base + j * _L + l16
            av = alive[sl]
            xv1 = x1v[sl]
            yv1 = y1v[sl]
            xv2 = x2v[sl]
            yv2 = y2v[sl]
            scv = sv_[sl]
            w = jnp.maximum(jnp.minimum(xv2, x2w) - jnp.maximum(xv1, x1w), 0.0)
            h = jnp.maximum(jnp.minimum(yv2, y2w) - jnp.maximum(yv1, y1w), 0.0)
            inter = w * h
            areav = (xv2 - xv1) * (yv2 - yv1)
            denom = jnp.maximum(areav + aw - inter, 1e-9)
            anew = jnp.where(
                (iv != ipv) & ~((inter > _T * denom) & vwv) & (av > 0.5),
                1.0, 0.0)
            alive[sl] = anew
            hi = (scv > sw) | ((scv == sw) & (iv < ipv))
            prv = prv + jnp.where(hi, 1.0, 0.0)
            cand = jnp.where(anew > 0.5, scv, _NEG)
            upd = cand > mv
            mivf = jnp.where(upd, iv.astype(jnp.float32), mivf)
            mv = jnp.maximum(mv, cand)
        mloc = jnp.max(mv)
        ilocf = jnp.min(jnp.where(mv == mloc, mivf, jnp.float32(1e9)))
        prf = jnp.sum(prv)
        li = jnp.clip(jnp.full((16,), ilocf.astype(jnp.int32)) - base,
                      0, _SLICE - 1)
        xc1 = plsc.load_gather(x1v, [li])
        yc1 = plsc.load_gather(y1v, [li])
        xc2 = plsc.load_gather(x2v, [li])
        yc2 = plsc.load_gather(y2v, [li])
        row = jnp.where(l16 == 0, mloc,
                        jnp.where(l16 == 1, ilocf,
                                  jnp.where(l16 == 2, prf,
                                            jnp.where(l16 == 3, xc1,
                                                      jnp.where(l16 == 4, yc1,
                                                                jnp.where(l16 == 5, xc2,
                                                                          jnp.where(l16 == 6, yc2,
                                                                                    0.0)))))))
        stage[...] = row
        p = t & 1
        pltpu.sync_copy(stage, cands.at[p, sid])
        plsc.subcore_barrier()
        pltpu.sync_copy(cands.at[p], candv)
        zi = jnp.zeros((16,), jnp.int32)
        sc16 = plsc.load_gather(candv, [l16, zi])
        ixf16 = plsc.load_gather(candv, [l16, zi + 1])
        pr16 = plsc.load_gather(candv, [l16, zi + 2])
        mg = jnp.max(sc16)
        winm = sc16 == mg
        iwf = jnp.min(jnp.where(winm, ixf16, jnp.float32(1e9)))
        iw = jnp.full((16,), iwf.astype(jnp.int32))
        rank = jnp.full((16,), jnp.sum(pr16).astype(jnp.int32))
        l16f = l16.astype(jnp.float32)
        wlf = jnp.min(jnp.where(winm & (ixf16 == iwf), l16f,
                                jnp.float32(1e9)))
        wlv = jnp.full((16,), wlf.astype(jnp.int32))
        tv = jnp.full((16,), t, jnp.int32)
        rec = vwv & (tv - 1 < cntvec) & (rank >= obase_v) & \
              (rank < obase_v + _OROWS)
        rrow = jnp.where(l16 == 0, x1w,
                         jnp.where(l16 == 1, y1w,
                                   jnp.where(l16 == 2, x2w,
                                             jnp.where(l16 == 3, y2w,
                                                       jnp.where(l16 == 4, sw,
                                                                 0.0)))))
        ridx = jnp.where(rec, (rank - obase_v) * _OW, 0) + l16
        plsc.store_scatter(outv, [ridx], rrow, mask=rec & (l16 < _OW))
        x1n = plsc.load_gather(candv, [wlv, zi + 3])
        y1n = plsc.load_gather(candv, [wlv, zi + 4])
        x2n = plsc.load_gather(candv, [wlv, zi + 5])
        y2n = plsc.load_gather(candv, [wlv, zi + 6])
        sn = jnp.full((16,), mg)
        an = (x2n - x1n) * (y2n - y1n)
        return iw, x1n, y1n, x2n, y2n, sn, an

    big = jnp.full((16,), 3e38, jnp.float32)
    bigi = jnp.full((16,), 2**30, jnp.int32)
    carry = (bigi, zeros16, zeros16, zeros16, zeros16, big, zeros16)
    lax.fori_loop(0, _MAX_ROUNDS + 1, round_body, carry)

    @pl.when(sid < _LAST)
    def _():
        pltpu.sync_copy(outv,
                        outh.at[pl.ds(sid * _OROWS * _OW, _OROWS * _OW)])

    @pl.when(sid == _LAST)
    def _():
        pltpu.sync_copy(outv.at[pl.ds(0, _LASTW)],
                        outh.at[pl.ds(_LAST * _OROWS * _OW, _LASTW)])


@jax.jit
def _nms_sc(boxes, scores, counts):
    bflat = jnp.pad(boxes.reshape(-1), (0, (_PAD - _N) * 4))
    s = jnp.pad(scores, (0, _PAD - _N))
    cnt = jnp.full((16,), jnp.asarray(counts, jnp.int32))
    mesh = plsc.VectorSubcoreMesh(core_axis_name="c", subcore_axis_name="s",
                                  num_cores=1)
    f = pl.kernel(
        _sc_body,
        out_type=jax.ShapeDtypeStruct((_N * _OW,), jnp.float32),
        mesh=mesh,
        compiler_params=pltpu.CompilerParams(needs_layout_passes=False,
                                             use_tc_tiling_on_sc=False),
        scratch_types=[
            pltpu.VMEM((_SLICE * 4,), jnp.float32),
            pltpu.VMEM((_SLICE,), jnp.float32),
            pltpu.VMEM((_SLICE,), jnp.float32),
            pltpu.VMEM((_SLICE,), jnp.float32),
            pltpu.VMEM((_SLICE,), jnp.float32),
            pltpu.VMEM((_SLICE,), jnp.float32),
            pltpu.VMEM((16,), jnp.int32),
            pltpu.VMEM((_SLICE,), jnp.float32),
            pltpu.VMEM((_OROWS * _OW,), jnp.float32),
            pltpu.VMEM((16,), jnp.float32),
            pltpu.VMEM((16, 16), jnp.float32),
            pltpu.VMEM_SHARED((2, 16, 16), jnp.float32),
        ],
    )
    out = f(bflat, s, cnt)
    return out.reshape(_N, _OW)


def kernel(boxes, scores, counts):
    return _nms_sc(boxes, scores, counts)


# speedup vs baseline: 1.3036x; 1.0006x over previous
import jax
import jax.numpy as jnp
from jax import lax
from jax.experimental import pallas as pl
from jax.experimental.pallas import tpu as pltpu
from jax.experimental.pallas import tpu_sc as plsc

_N = 5000
_PAD = 5120
_T = 0.5
_MAX_ROUNDS = 100
_NEG = -1e30
_NSUB = 16
_L = 16
_SLICE = _PAD // _NSUB
_CHUNKS = _SLICE // _L
_OROWS = _PAD // _NSUB
_OW = 5
_LAST = _NSUB - 1
_LASTW = (_N - _LAST * _OROWS) * _OW


def _sc_body(bh, sh, cnth, outh, bv, x1v, y1v, x2v, y2v, sv_, cntv, alive,
             outv, stage, candv, cands):
    sid = lax.axis_index("s")
    base = sid * _SLICE
    l16 = lax.iota(jnp.int32, 16)
    l16x4 = l16 * 4
    zeros16 = jnp.zeros((16,), jnp.float32)
    obase_v = jnp.full((16,), sid * _OROWS, jnp.int32)

    pltpu.sync_copy(bh.at[pl.ds(base * 4, _SLICE * 4)], bv)
    pltpu.sync_copy(sh.at[pl.ds(base, _SLICE)], sv_)
    pltpu.sync_copy(cnth, cntv)
    cntvec = cntv[...]

    for j in range(_CHUNKS):
        sl = pl.ds(j * _L, _L)
        bix = j * 64 + l16x4
        x1v[sl] = plsc.load_gather(bv, [bix])
        y1v[sl] = plsc.load_gather(bv, [bix + 1])
        x2v[sl] = plsc.load_gather(bv, [bix + 2])
        y2v[sl] = plsc.load_gather(bv, [bix + 3])

    for j in range(_CHUNKS):
        iv = base + j * _L + l16
        alive[pl.ds(j * _L, _L)] = jnp.where(iv < _N, 1.0, 0.0)
    for r in range(_OROWS * _OW // _L):
        outv[pl.ds(r * _L, _L)] = zeros16

    def round_body(t, carry):
        ipv, x1w, y1w, x2w, y2w, sw, aw = carry
        vwv = ipv < _N
        mv = jnp.full((16,), _NEG, jnp.float32)
        mivf = jnp.full((16,), 1e9, jnp.float32)
        prv = zeros16
        for j in range(_CHUNKS):
            sl = pl.ds(j * _L, _L)
            iv = base + j * _L + l16
            av = alive[sl]
            xv1 = x1v[sl]
            yv1 = y1v[sl]
            xv2 = x2v[sl]
            yv2 = y2v[sl]
            scv = sv_[sl]
            w = jnp.maximum(jnp.minimum(xv2, x2w) - jnp.maximum(xv1, x1w), 0.0)
            h = jnp.maximum(jnp.minimum(yv2, y2w) - jnp.maximum(yv1, y1w), 0.0)
            inter = w * h
            areav = (xv2 - xv1) * (yv2 - yv1)
            denom = jnp.maximum(areav + aw - inter, 1e-9)
            anew = jnp.where(
                (iv != ipv) & ~((inter > _T * denom) & vwv) & (av > 0.5),
                1.0, 0.0)
            alive[sl] = anew
            hi = (scv > sw) | ((scv == sw) & (iv < ipv))
            prv = prv + jnp.where(hi, 1.0, 0.0)
            cand = jnp.where(anew > 0.5, scv, _NEG)
            upd = cand > mv
            mivf = jnp.where(upd, iv.astype(jnp.float32), mivf)
            mv = jnp.maximum(mv, cand)
        mloc = jnp.max(mv)
        ilocf = jnp.min(jnp.where(mv == mloc, mivf, jnp.float32(1e9)))
        prf = jnp.sum(prv)
        li = jnp.clip(jnp.full((16,), ilocf.astype(jnp.int32)) - base,
                      0, _SLICE - 1)
        xc1 = plsc.load_gather(x1v, [li])
        yc1 = plsc.load_gather(y1v, [li])
        xc2 = plsc.load_gather(x2v, [li])
        yc2 = plsc.load_gather(y2v, [li])
        row = jnp.where(l16 == 0, mloc,
                        jnp.where(l16 == 1, ilocf,
                                  jnp.where(l16 == 2, prf,
                                            jnp.where(l16 == 3, xc1,
                                                      jnp.where(l16 == 4, yc1,
                                                                jnp.where(l16 == 5, xc2,
                                                                          jnp.where(l16 == 6, yc2,
                                                                                    0.0)))))))
        stage[...] = row
        p = t & 1
        pltpu.sync_copy(stage, cands.at[p, sid])
        plsc.subcore_barrier()
        pltpu.sync_copy(cands.at[p], candv)
        zi = jnp.zeros((16,), jnp.int32)
        sc16 = plsc.load_gather(candv, [l16, zi])
        ixf16 = plsc.load_gather(candv, [l16, zi + 1])
        pr16 = plsc.load_gather(candv, [l16, zi + 2])
        mg = jnp.max(sc16)
        winm = sc16 == mg
        iwf = jnp.min(jnp.where(winm, ixf16, jnp.float32(1e9)))
        iw = jnp.full((16,), iwf.astype(jnp.int32))
        rank = jnp.full((16,), jnp.sum(pr16).astype(jnp.int32))
        l16f = l16.astype(jnp.float32)
        wlf = jnp.min(jnp.where(winm & (ixf16 == iwf), l16f,
                                jnp.float32(1e9)))
        wlv = jnp.full((16,), wlf.astype(jnp.int32))
        tv = jnp.full((16,), t, jnp.int32)
        rec = vwv & (tv - 1 < cntvec) & (rank >= obase_v) & \
              (rank < obase_v + _OROWS)
        rrow = jnp.where(l16 == 0, x1w,
                         jnp.where(l16 == 1, y1w,
                                   jnp.where(l16 == 2, x2w,
                                             jnp.where(l16 == 3, y2w,
                                                       jnp.where(l16 == 4, sw,
                                                                 0.0)))))
        ridx = jnp.where(rec, (rank - obase_v) * _OW, 0) + l16
        plsc.store_scatter(outv, [ridx], rrow, mask=rec & (l16 < _OW))
        x1n = plsc.load_gather(candv, [wlv, zi + 3])
        y1n = plsc.load_gather(candv, [wlv, zi + 4])
        x2n = plsc.load_gather(candv, [wlv, zi + 5])
        y2n = plsc.load_gather(candv, [wlv, zi + 6])
        sn = jnp.full((16,), mg)
        an = (x2n - x1n) * (y2n - y1n)
        return iw, x1n, y1n, x2n, y2n, sn, an

    big = jnp.full((16,), 3e38, jnp.float32)
    bigi = jnp.full((16,), 2**30, jnp.int32)
    carry = (bigi, zeros16, zeros16, zeros16, zeros16, big, zeros16)
    lax.fori_loop(0, _MAX_ROUNDS + 1, round_body, carry)

    @pl.when(sid < _LAST)
    def _():
        pltpu.sync_copy(outv,
                        outh.at[pl.ds(sid * _OROWS * _OW, _OROWS * _OW)])

    @pl.when(sid == _LAST)
    def _():
        pltpu.sync_copy(outv.at[pl.ds(0, _LASTW)],
                        outh.at[pl.ds(_LAST * _OROWS * _OW, _LASTW)])


@jax.jit
def _nms_sc(boxes, scores, counts):
    bflat = jnp.pad(boxes.reshape(-1), (0, (_PAD - _N) * 4))
    s = jnp.pad(scores, (0, _PAD - _N))
    cnt = jnp.full((16,), jnp.asarray(counts, jnp.int32))
    mesh = plsc.VectorSubcoreMesh(core_axis_name="c", subcore_axis_name="s",
                                  num_cores=1)
    f = pl.kernel(
        _sc_body,
        out_type=jax.ShapeDtypeStruct((_N * _OW,), jnp.float32),
        mesh=mesh,
        compiler_params=pltpu.CompilerParams(needs_layout_passes=False,
                                             use_tc_tiling_on_sc=False,
                                             disable_bounds_checks=True,
                                             disable_semaphore_checks=True,
                                             skip_device_barrier=True),
        scratch_types=[
            pltpu.VMEM((_SLICE * 4,), jnp.float32),
            pltpu.VMEM((_SLICE,), jnp.float32),
            pltpu.VMEM((_SLICE,), jnp.float32),
            pltpu.VMEM((_SLICE,), jnp.float32),
            pltpu.VMEM((_SLICE,), jnp.float32),
            pltpu.VMEM((_SLICE,), jnp.float32),
            pltpu.VMEM((16,), jnp.int32),
            pltpu.VMEM((_SLICE,), jnp.float32),
            pltpu.VMEM((_OROWS * _OW,), jnp.float32),
            pltpu.VMEM((16,), jnp.float32),
            pltpu.VMEM((16, 16), jnp.float32),
            pltpu.VMEM_SHARED((2, 16, 16), jnp.float32),
        ],
    )
    out = f(bflat, s, cnt)
    return out.reshape(_N, _OW)


def kernel(boxes, scores, counts):
    return _nms_sc(boxes, scores, counts)


# speedup vs baseline: 1.4682x; 1.1262x over previous
import jax
import jax.numpy as jnp
from jax import lax
from jax.experimental import pallas as pl
from jax.experimental.pallas import tpu as pltpu
from jax.experimental.pallas import tpu_sc as plsc

_N = 5000
_PAD = 5120
_T = 0.5
_MAX_ROUNDS = 100
_NEG = -1e30
_BIGS = 3e38
_NSUB = 16
_L = 16
_SLICE = _PAD // _NSUB
_CHUNKS = _SLICE // _L
_OROWS = _PAD // _NSUB
_OW = 5
_LAST = _NSUB - 1
_LASTW = (_N - _LAST * _OROWS) * _OW


def _sc_body(bh, sh, cnth, outh, bv, x1v, y1v, x2v, y2v, sv_, cntv, alive,
             outv, stage, candv, cands):
    sid = lax.axis_index("s")
    base = sid * _SLICE
    l16 = lax.iota(jnp.int32, 16)
    l16f = l16.astype(jnp.float32)
    l16x4 = l16 * 4
    zeros16 = jnp.zeros((16,), jnp.float32)
    obase_v = jnp.full((16,), sid * _OROWS, jnp.int32)

    pltpu.sync_copy(bh.at[pl.ds(base * 4, _SLICE * 4)], bv)
    pltpu.sync_copy(sh.at[pl.ds(base, _SLICE)], sv_)
    pltpu.sync_copy(cnth, cntv)
    cnt = jnp.max(cntv[...].astype(jnp.float32)).astype(jnp.int32)

    for j in range(_CHUNKS):
        sl = pl.ds(j * _L, _L)
        bix = j * 64 + l16x4
        x1v[sl] = plsc.load_gather(bv, [bix])
        y1v[sl] = plsc.load_gather(bv, [bix + 1])
        x2v[sl] = plsc.load_gather(bv, [bix + 2])
        y2v[sl] = plsc.load_gather(bv, [bix + 3])

    for j in range(_CHUNKS):
        iv = base + j * _L + l16
        alive[pl.ds(j * _L, _L)] = jnp.where(iv < _N, 1.0, 0.0)
    for r in range(_OROWS * _OW // _L):
        outv[pl.ds(r * _L, _L)] = zeros16

    def cond_fn(carry):
        t, kc, exh, ip1s, ip2s = carry[:5]
        pend = (ip1s < jnp.float32(_N)) | (ip2s < jnp.float32(_N))
        more = (kc < cnt) & (t <= _MAX_ROUNDS) & (exh == 0)
        return pend | more

    def round_body(carry):
        (t, kc, exh, ip1s, ip2s,
         ip1v, x11, y11, x21, y21, s1v, a1v,
         ip2v, x12, y12, x22, y22, s2v, a2v) = carry
        vw1 = ip1v < _N
        vw2 = ip2v < _N
        mv1 = jnp.full((16,), _NEG, jnp.float32)
        mi1 = jnp.full((16,), 1e9, jnp.float32)
        mv2 = jnp.full((16,), _NEG, jnp.float32)
        mi2 = jnp.full((16,), 1e9, jnp.float32)
        pr1 = zeros16
        pr2 = zeros16
        for j in range(_CHUNKS):
            sl = pl.ds(j * _L, _L)
            iv = base + j * _L + l16
            av = alive[sl]
            xv1 = x1v[sl]
            yv1 = y1v[sl]
            xv2 = x2v[sl]
            yv2 = y2v[sl]
            scv = sv_[sl]
            areav = (xv2 - xv1) * (yv2 - yv1)
            w1 = jnp.maximum(jnp.minimum(xv2, x21) - jnp.maximum(xv1, x11),
                             0.0)
            h1 = jnp.maximum(jnp.minimum(yv2, y21) - jnp.maximum(yv1, y11),
                             0.0)
            in1 = w1 * h1
            k1 = (in1 > _T * jnp.maximum(areav + a1v - in1, 1e-9)) & vw1
            w2 = jnp.maximum(jnp.minimum(xv2, x22) - jnp.maximum(xv1, x12),
                             0.0)
            h2 = jnp.maximum(jnp.minimum(yv2, y22) - jnp.maximum(yv1, y12),
                             0.0)
            in2 = w2 * h2
            k2 = (in2 > _T * jnp.maximum(areav + a2v - in2, 1e-9)) & vw2
            anew = jnp.where((iv != ip1v) & (iv != ip2v) & ~k1 & ~k2 &
                             (av > 0.5), 1.0, 0.0)
            alive[sl] = anew
            hi1 = (scv > s1v) | ((scv == s1v) & (iv < ip1v))
            pr1 = pr1 + jnp.where(hi1, 1.0, 0.0)
            hi2 = (scv > s2v) | ((scv == s2v) & (iv < ip2v))
            pr2 = pr2 + jnp.where(hi2, 1.0, 0.0)
            c = jnp.where(anew > 0.5, scv, _NEG)
            ivf = iv.astype(jnp.float32)
            g1 = c > mv1
            g2 = c > mv2
            mv2 = jnp.where(g1, mv1, jnp.where(g2, c, mv2))
            mi2 = jnp.where(g1, mi1, jnp.where(g2, ivf, mi2))
            mv1 = jnp.where(g1, c, mv1)
            mi1 = jnp.where(g1, ivf, mi1)
        m1 = jnp.max(mv1)
        i1f = jnp.min(jnp.where(mv1 == m1, mi1, jnp.float32(1e9)))
        i1b = jnp.full((16,), i1f)
        sc2l = jnp.where(mi1 == i1b, mv2, mv1)
        id2l = jnp.where(mi1 == i1b, mi2, mi1)
        m2 = jnp.max(sc2l)
        i2f = jnp.min(jnp.where(sc2l == m2, id2l, jnp.float32(1e9)))
        p1s = jnp.sum(pr1)
        p2s = jnp.sum(pr2)
        li1 = jnp.clip(jnp.full((16,), i1f.astype(jnp.int32)) - base,
                       0, _SLICE - 1)
        li2 = jnp.clip(jnp.full((16,), i2f.astype(jnp.int32)) - base,
                       0, _SLICE - 1)
        ax1 = plsc.load_gather(x1v, [li1])
        ay1 = plsc.load_gather(y1v, [li1])
        ax2 = plsc.load_gather(x2v, [li1])
        ay2 = plsc.load_gather(y2v, [li1])
        bx1 = plsc.load_gather(x1v, [li2])
        by1 = plsc.load_gather(y1v, [li2])
        bx2 = plsc.load_gather(x2v, [li2])
        by2 = plsc.load_gather(y2v, [li2])
        row = jnp.where(l16 == 0, m1,
              jnp.where(l16 == 1, i1f,
              jnp.where(l16 == 2, m2,
              jnp.where(l16 == 3, i2f,
              jnp.where(l16 == 4, p1s,
              jnp.where(l16 == 5, p2s,
              jnp.where(l16 == 6, ax1,
              jnp.where(l16 == 7, ay1,
              jnp.where(l16 == 8, ax2,
              jnp.where(l16 == 9, ay2,
              jnp.where(l16 == 10, bx1,
              jnp.where(l16 == 11, by1,
              jnp.where(l16 == 12, bx2,
              jnp.where(l16 == 13, by2, 0.0))))))))))))))
        stage[...] = row
        p = t & 1
        pltpu.sync_copy(stage, cands.at[p, sid])
        plsc.subcore_barrier()
        pltpu.sync_copy(cands.at[p], candv)
        zi = jnp.zeros((16,), jnp.int32)
        a_s = plsc.load_gather(candv, [l16, zi])
        a_i = plsc.load_gather(candv, [l16, zi + 1])
        b_s = plsc.load_gather(candv, [l16, zi + 2])
        b_i = plsc.load_gather(candv, [l16, zi + 3])
        a_p = plsc.load_gather(candv, [l16, zi + 4])
        b_p = plsc.load_gather(candv, [l16, zi + 5])
        mg1 = jnp.max(a_s)
        iw1f = jnp.min(jnp.where(a_s == mg1, a_i, jnp.float32(1e9)))
        wl1f = jnp.min(jnp.where((a_s == mg1) & (a_i == jnp.full((16,), iw1f)),
                                 l16f, jnp.float32(1e9)))
        wl1v = jnp.full((16,), wl1f.astype(jnp.int32))
        nx11 = plsc.load_gather(candv, [wl1v, zi + 6])
        ny11 = plsc.load_gather(candv, [wl1v, zi + 7])
        nx21 = plsc.load_gather(candv, [wl1v, zi + 8])
        ny21 = plsc.load_gather(candv, [wl1v, zi + 9])
        sc2m = jnp.where(l16 == wl1v, b_s, a_s)
        ix2m = jnp.where(l16 == wl1v, b_i, a_i)
        mg2 = jnp.max(sc2m)
        iw2f = jnp.min(jnp.where(sc2m == mg2, ix2m, jnp.float32(1e9)))
        wl2f = jnp.min(jnp.where((sc2m == mg2) & (ix2m == jnp.full((16,), iw2f)),
                                 l16f, jnp.float32(1e9)))
        wl2v = jnp.full((16,), wl2f.astype(jnp.int32))
        cb = jnp.where(wl2v == wl1v, zi + 10, zi + 6)
        nx12 = plsc.load_gather(candv, [wl2v, cb])
        ny12 = plsc.load_gather(candv, [wl2v, cb + 1])
        nx22 = plsc.load_gather(candv, [wl2v, cb + 2])
        ny22 = plsc.load_gather(candv, [wl2v, cb + 3])
        rank1 = jnp.full((16,), jnp.sum(a_p).astype(jnp.int32))
        rank2 = jnp.full((16,), jnp.sum(b_p).astype(jnp.int32))
        rec1 = vw1 & (rank1 >= obase_v) & (rank1 < obase_v + _OROWS)
        rr1 = jnp.where(l16 == 0, x11,
              jnp.where(l16 == 1, y11,
              jnp.where(l16 == 2, x21,
              jnp.where(l16 == 3, y21,
              jnp.where(l16 == 4, s1v, 0.0)))))
        ri1 = jnp.where(rec1, (rank1 - obase_v) * _OW, 0) + l16
        plsc.store_scatter(outv, [ri1], rr1, mask=rec1 & (l16 < _OW))
        rec2 = vw2 & (rank2 >= obase_v) & (rank2 < obase_v + _OROWS)
        rr2 = jnp.where(l16 == 0, x12,
              jnp.where(l16 == 1, y12,
              jnp.where(l16 == 2, x22,
              jnp.where(l16 == 3, y22,
              jnp.where(l16 == 4, s2v, 0.0)))))
        ri2 = jnp.where(rec2, (rank2 - obase_v) * _OW, 0) + l16
        plsc.store_scatter(outv, [ri2], rr2, mask=rec2 & (l16 < _OW))
        acc1 = (iw1f < jnp.float32(_N)) & (kc < cnt) & (t <= _MAX_ROUNDS)
        inw = jnp.maximum(jnp.minimum(nx21, nx22) - jnp.maximum(nx11, nx12),
                          0.0)
        inh = jnp.maximum(jnp.minimum(ny21, ny22) - jnp.maximum(ny11, ny12),
                          0.0)
        in12 = inw * inh
        ar1 = (nx21 - nx11) * (ny21 - ny11)
        ar2 = (nx22 - nx12) * (ny22 - ny12)
        sepv = in12 <= _T * jnp.maximum(ar1 + ar2 - in12, 1e-9)
        sep = jnp.max(jnp.where(sepv, 1.0, 0.0)) > 0.5
        acc2 = acc1 & (iw2f < jnp.float32(_N)) & sep & (kc + 1 < cnt)
        kc_n = kc + jnp.where(acc1, 1, 0) + jnp.where(acc2, 1, 0)
        exh_n = jnp.where(iw1f < jnp.float32(_N), 0, 1)
        bigiv = jnp.full((16,), 2**30, jnp.int32)
        bigsv = jnp.full((16,), _BIGS, jnp.float32)
        ip1v_n = jnp.where(acc1, jnp.full((16,), iw1f.astype(jnp.int32)),
                           bigiv)
        s1v_n = jnp.where(acc1, jnp.full((16,), mg1), bigsv)
        a1v_n = ar1
        ip2v_n = jnp.where(acc2, jnp.full((16,), iw2f.astype(jnp.int32)),
                           bigiv)
        s2v_n = jnp.where(acc2, jnp.full((16,), mg2), bigsv)
        a2v_n = ar2
        ip1s_n = jnp.where(acc1, iw1f, jnp.float32(1e9))
        ip2s_n = jnp.where(acc2, iw2f, jnp.float32(1e9))
        return (t + 1, kc_n, exh_n, ip1s_n, ip2s_n,
                ip1v_n, nx11, ny11, nx21, ny21, s1v_n, a1v_n,
                ip2v_n, nx12, ny12, nx22, ny22, s2v_n, a2v_n)

    big = jnp.full((16,), _BIGS, jnp.float32)
    bigi = jnp.full((16,), 2**30, jnp.int32)
    carry = (jnp.int32(0), jnp.int32(0), jnp.int32(0),
             jnp.float32(1e9), jnp.float32(1e9),
             bigi, zeros16, zeros16, zeros16, zeros16, big, zeros16,
             bigi, zeros16, zeros16, zeros16, zeros16, big, zeros16)
    lax.while_loop(cond_fn, round_body, carry)

    @pl.when(sid < _LAST)
    def _():
        pltpu.sync_copy(outv,
                        outh.at[pl.ds(sid * _OROWS * _OW, _OROWS * _OW)])

    @pl.when(sid == _LAST)
    def _():
        pltpu.sync_copy(outv.at[pl.ds(0, _LASTW)],
                        outh.at[pl.ds(_LAST * _OROWS * _OW, _LASTW)])


@jax.jit
def _nms_sc(boxes, scores, counts):
    bflat = jnp.pad(boxes.reshape(-1), (0, (_PAD - _N) * 4))
    s = jnp.pad(scores, (0, _PAD - _N))
    cnt = jnp.full((16,), jnp.asarray(counts, jnp.int32))
    mesh = plsc.VectorSubcoreMesh(core_axis_name="c", subcore_axis_name="s",
                                  num_cores=1)
    f = pl.kernel(
        _sc_body,
        out_type=jax.ShapeDtypeStruct((_N * _OW,), jnp.float32),
        mesh=mesh,
        compiler_params=pltpu.CompilerParams(needs_layout_passes=False,
                                             use_tc_tiling_on_sc=False),
        scratch_types=[
            pltpu.VMEM((_SLICE * 4,), jnp.float32),
            pltpu.VMEM((_SLICE,), jnp.float32),
            pltpu.VMEM((_SLICE,), jnp.float32),
            pltpu.VMEM((_SLICE,), jnp.float32),
            pltpu.VMEM((_SLICE,), jnp.float32),
            pltpu.VMEM((_SLICE,), jnp.float32),
            pltpu.VMEM((16,), jnp.int32),
            pltpu.VMEM((_SLICE,), jnp.float32),
            pltpu.VMEM((_OROWS * _OW,), jnp.float32),
            pltpu.VMEM((16,), jnp.float32),
            pltpu.VMEM((16, 16), jnp.float32),
            pltpu.VMEM_SHARED((2, 16, 16), jnp.float32),
        ],
    )
    out = f(bflat, s, cnt)
    return out.reshape(_N, _OW)


def kernel(boxes, scores, counts):
    return _nms_sc(boxes, scores, counts)


# speedup vs baseline: 1.5544x; 1.0587x over previous
import jax
import jax.numpy as jnp
from jax import lax
from jax.experimental import pallas as pl
from jax.experimental.pallas import tpu as pltpu
from jax.experimental.pallas import tpu_sc as plsc

_N = 5000
_PAD = 5120
_T = 0.5
_MAX_ROUNDS = 100
_NEG = -1e30
_BIGS = 3e38
_NSUB = 16
_L = 16
_SLICE = _PAD // _NSUB
_CHUNKS = _SLICE // _L
_OROWS = _PAD // _NSUB
_OW = 5
_LAST = _NSUB - 1
_LASTW = (_N - _LAST * _OROWS) * _OW


def _sc_body(bh, sh, cnth, outh, bv, x1v, y1v, x2v, y2v, sv_, cntv, alive,
             outv, stage, candv, cands):
    sid = lax.axis_index("s")
    base = sid * _SLICE
    l16 = lax.iota(jnp.int32, 16)
    l16f = l16.astype(jnp.float32)
    l16x4 = l16 * 4
    zeros16 = jnp.zeros((16,), jnp.float32)
    obase_v = jnp.full((16,), sid * _OROWS, jnp.int32)

    pltpu.sync_copy(bh.at[pl.ds(base * 4, _SLICE * 4)], bv)
    pltpu.sync_copy(sh.at[pl.ds(base, _SLICE)], sv_)
    pltpu.sync_copy(cnth, cntv)
    cnt = jnp.max(cntv[...].astype(jnp.float32)).astype(jnp.int32)

    for j in range(_CHUNKS):
        sl = pl.ds(j * _L, _L)
        bix = j * 64 + l16x4
        x1v[sl] = plsc.load_gather(bv, [bix])
        y1v[sl] = plsc.load_gather(bv, [bix + 1])
        x2v[sl] = plsc.load_gather(bv, [bix + 2])
        y2v[sl] = plsc.load_gather(bv, [bix + 3])

    for j in range(_CHUNKS):
        iv = base + j * _L + l16
        alive[pl.ds(j * _L, _L)] = jnp.where(iv < _N, 1.0, 0.0)
    for r in range(_OROWS * _OW // _L):
        outv[pl.ds(r * _L, _L)] = zeros16

    def cond_fn(carry):
        t, kc, exh, ip1s, ip2s = carry[:5]
        pend = (ip1s < jnp.float32(_N)) | (ip2s < jnp.float32(_N))
        more = (kc < cnt) & (t <= _MAX_ROUNDS) & (exh == 0)
        return pend | more

    def round_body(carry):
        (t, kc, exh, ip1s, ip2s,
         ip1v, x11, y11, x21, y21, s1v, a1v,
         ip2v, x12, y12, x22, y22, s2v, a2v) = carry
        vw1 = ip1v < _N
        vw2 = ip2v < _N
        mv1 = jnp.full((16,), _NEG, jnp.float32)
        mi1 = jnp.full((16,), 1e9, jnp.float32)
        mv2 = jnp.full((16,), _NEG, jnp.float32)
        mi2 = jnp.full((16,), 1e9, jnp.float32)
        pr1 = zeros16
        pr2 = zeros16
        for j in range(_CHUNKS):
            sl = pl.ds(j * _L, _L)
            iv = base + j * _L + l16
            av = alive[sl]
            xv1 = x1v[sl]
            yv1 = y1v[sl]
            xv2 = x2v[sl]
            yv2 = y2v[sl]
            scv = sv_[sl]
            areav = (xv2 - xv1) * (yv2 - yv1)
            w1 = jnp.maximum(jnp.minimum(xv2, x21) - jnp.maximum(xv1, x11),
                             0.0)
            h1 = jnp.maximum(jnp.minimum(yv2, y21) - jnp.maximum(yv1, y11),
                             0.0)
            in1 = w1 * h1
            k1 = (in1 > _T * jnp.maximum(areav + a1v - in1, 1e-9)) & vw1
            w2 = jnp.maximum(jnp.minimum(xv2, x22) - jnp.maximum(xv1, x12),
                             0.0)
            h2 = jnp.maximum(jnp.minimum(yv2, y22) - jnp.maximum(yv1, y12),
                             0.0)
            in2 = w2 * h2
            k2 = (in2 > _T * jnp.maximum(areav + a2v - in2, 1e-9)) & vw2
            anew = jnp.where((iv != ip1v) & (iv != ip2v) & ~k1 & ~k2 &
                             (av > 0.5), 1.0, 0.0)
            alive[sl] = anew
            hi1 = (scv > s1v) | ((scv == s1v) & (iv < ip1v))
            pr1 = pr1 + jnp.where(hi1, 1.0, 0.0)
            hi2 = (scv > s2v) | ((scv == s2v) & (iv < ip2v))
            pr2 = pr2 + jnp.where(hi2, 1.0, 0.0)
            c = jnp.where(anew > 0.5, scv, _NEG)
            ivf = iv.astype(jnp.float32)
            g1 = c > mv1
            g2 = c > mv2
            mv2 = jnp.where(g1, mv1, jnp.where(g2, c, mv2))
            mi2 = jnp.where(g1, mi1, jnp.where(g2, ivf, mi2))
            mv1 = jnp.where(g1, c, mv1)
            mi1 = jnp.where(g1, ivf, mi1)
        m1 = jnp.max(mv1)
        i1f = jnp.min(jnp.where(mv1 == m1, mi1, jnp.float32(1e9)))
        i1b = jnp.full((16,), i1f)
        sc2l = jnp.where(mi1 == i1b, mv2, mv1)
        id2l = jnp.where(mi1 == i1b, mi2, mi1)
        m2 = jnp.max(sc2l)
        i2f = jnp.min(jnp.where(sc2l == m2, id2l, jnp.float32(1e9)))
        p1s = jnp.sum(pr1)
        p2s = jnp.sum(pr2)
        li1 = jnp.clip(jnp.full((16,), i1f.astype(jnp.int32)) - base,
                       0, _SLICE - 1)
        li2 = jnp.clip(jnp.full((16,), i2f.astype(jnp.int32)) - base,
                       0, _SLICE - 1)
        ax1 = plsc.load_gather(x1v, [li1])
        ay1 = plsc.load_gather(y1v, [li1])
        ax2 = plsc.load_gather(x2v, [li1])
        ay2 = plsc.load_gather(y2v, [li1])
        bx1 = plsc.load_gather(x1v, [li2])
        by1 = plsc.load_gather(y1v, [li2])
        bx2 = plsc.load_gather(x2v, [li2])
        by2 = plsc.load_gather(y2v, [li2])
        row = jnp.where(l16 == 0, m1,
              jnp.where(l16 == 1, i1f,
              jnp.where(l16 == 2, m2,
              jnp.where(l16 == 3, i2f,
              jnp.where(l16 == 4, p1s,
              jnp.where(l16 == 5, p2s,
              jnp.where(l16 == 6, ax1,
              jnp.where(l16 == 7, ay1,
              jnp.where(l16 == 8, ax2,
              jnp.where(l16 == 9, ay2,
              jnp.where(l16 == 10, bx1,
              jnp.where(l16 == 11, by1,
              jnp.where(l16 == 12, bx2,
              jnp.where(l16 == 13, by2, 0.0))))))))))))))
        stage[...] = row
        p = t & 1
        pltpu.sync_copy(stage, cands.at[p, sid])
        plsc.subcore_barrier()
        pltpu.sync_copy(cands.at[p], candv)
        zi = jnp.zeros((16,), jnp.int32)
        a_s = plsc.load_gather(candv, [l16, zi])
        a_i = plsc.load_gather(candv, [l16, zi + 1])
        b_s = plsc.load_gather(candv, [l16, zi + 2])
        b_i = plsc.load_gather(candv, [l16, zi + 3])
        a_p = plsc.load_gather(candv, [l16, zi + 4])
        b_p = plsc.load_gather(candv, [l16, zi + 5])
        mg1 = jnp.max(a_s)
        wl1v = plsc.all_reduce_ffs(a_s == mg1)
        iw1f = jnp.max(plsc.load_gather(candv, [wl1v, zi + 1]))
        nx11 = plsc.load_gather(candv, [wl1v, zi + 6])
        ny11 = plsc.load_gather(candv, [wl1v, zi + 7])
        nx21 = plsc.load_gather(candv, [wl1v, zi + 8])
        ny21 = plsc.load_gather(candv, [wl1v, zi + 9])
        sc2m = jnp.where(l16 == wl1v, b_s, a_s)
        ix2m = jnp.where(l16 == wl1v, b_i, a_i)
        mg2 = jnp.max(sc2m)
        wl2v = plsc.all_reduce_ffs(sc2m == mg2)
        iw2f = jnp.max(jnp.where(l16 == wl2v, ix2m, jnp.float32(-1)))
        cb = jnp.where(wl2v == wl1v, zi + 10, zi + 6)
        nx12 = plsc.load_gather(candv, [wl2v, cb])
        ny12 = plsc.load_gather(candv, [wl2v, cb + 1])
        nx22 = plsc.load_gather(candv, [wl2v, cb + 2])
        ny22 = plsc.load_gather(candv, [wl2v, cb + 3])
        rank1 = jnp.full((16,), jnp.sum(a_p).astype(jnp.int32))
        rank2 = jnp.full((16,), jnp.sum(b_p).astype(jnp.int32))
        rec1 = vw1 & (rank1 >= obase_v) & (rank1 < obase_v + _OROWS)
        rr1 = jnp.where(l16 == 0, x11,
              jnp.where(l16 == 1, y11,
              jnp.where(l16 == 2, x21,
              jnp.where(l16 == 3, y21,
              jnp.where(l16 == 4, s1v, 0.0)))))
        ri1 = jnp.where(rec1, (rank1 - obase_v) * _OW, 0) + l16
        plsc.store_scatter(outv, [ri1], rr1, mask=rec1 & (l16 < _OW))
        rec2 = vw2 & (rank2 >= obase_v) & (rank2 < obase_v + _OROWS)
        rr2 = jnp.where(l16 == 0, x12,
              jnp.where(l16 == 1, y12,
              jnp.where(l16 == 2, x22,
              jnp.where(l16 == 3, y22,
              jnp.where(l16 == 4, s2v, 0.0)))))
        ri2 = jnp.where(rec2, (rank2 - obase_v) * _OW, 0) + l16
        plsc.store_scatter(outv, [ri2], rr2, mask=rec2 & (l16 < _OW))
        acc1 = (iw1f < jnp.float32(_N)) & (kc < cnt) & (t <= _MAX_ROUNDS)
        inw = jnp.maximum(jnp.minimum(nx21, nx22) - jnp.maximum(nx11, nx12),
                          0.0)
        inh = jnp.maximum(jnp.minimum(ny21, ny22) - jnp.maximum(ny11, ny12),
                          0.0)
        in12 = inw * inh
        ar1 = (nx21 - nx11) * (ny21 - ny11)
        ar2 = (nx22 - nx12) * (ny22 - ny12)
        sepv = in12 <= _T * jnp.maximum(ar1 + ar2 - in12, 1e-9)
        sep = jnp.max(jnp.where(sepv, 1.0, 0.0)) > 0.5
        acc2 = acc1 & (iw2f < jnp.float32(_N)) & sep & (kc + 1 < cnt)
        kc_n = kc + jnp.where(acc1, 1, 0) + jnp.where(acc2, 1, 0)
        exh_n = jnp.where(iw1f < jnp.float32(_N), 0, 1)
        bigiv = jnp.full((16,), 2**30, jnp.int32)
        bigsv = jnp.full((16,), _BIGS, jnp.float32)
        ip1v_n = jnp.where(acc1, jnp.full((16,), iw1f.astype(jnp.int32)),
                           bigiv)
        s1v_n = jnp.where(acc1, jnp.full((16,), mg1), bigsv)
        a1v_n = ar1
        ip2v_n = jnp.where(acc2, jnp.full((16,), iw2f.astype(jnp.int32)),
                           bigiv)
        s2v_n = jnp.where(acc2, jnp.full((16,), mg2), bigsv)
        a2v_n = ar2
        ip1s_n = jnp.where(acc1, iw1f, jnp.float32(1e9))
        ip2s_n = jnp.where(acc2, iw2f, jnp.float32(1e9))
        return (t + 1, kc_n, exh_n, ip1s_n, ip2s_n,
                ip1v_n, nx11, ny11, nx21, ny21, s1v_n, a1v_n,
                ip2v_n, nx12, ny12, nx22, ny22, s2v_n, a2v_n)

    big = jnp.full((16,), _BIGS, jnp.float32)
    bigi = jnp.full((16,), 2**30, jnp.int32)
    carry = (jnp.int32(0), jnp.int32(0), jnp.int32(0),
             jnp.float32(1e9), jnp.float32(1e9),
             bigi, zeros16, zeros16, zeros16, zeros16, big, zeros16,
             bigi, zeros16, zeros16, zeros16, zeros16, big, zeros16)
    lax.while_loop(cond_fn, round_body, carry)

    @pl.when(sid < _LAST)
    def _():
        pltpu.sync_copy(outv,
                        outh.at[pl.ds(sid * _OROWS * _OW, _OROWS * _OW)])

    @pl.when(sid == _LAST)
    def _():
        pltpu.sync_copy(outv.at[pl.ds(0, _LASTW)],
                        outh.at[pl.ds(_LAST * _OROWS * _OW, _LASTW)])


@jax.jit
def _nms_sc(boxes, scores, counts):
    bflat = jnp.pad(boxes.reshape(-1), (0, (_PAD - _N) * 4))
    s = jnp.pad(scores, (0, _PAD - _N))
    cnt = jnp.full((16,), jnp.asarray(counts, jnp.int32))
    mesh = plsc.VectorSubcoreMesh(core_axis_name="c", subcore_axis_name="s",
                                  num_cores=1)
    f = pl.kernel(
        _sc_body,
        out_type=jax.ShapeDtypeStruct((_N * _OW,), jnp.float32),
        mesh=mesh,
        compiler_params=pltpu.CompilerParams(needs_layout_passes=False,
                                             use_tc_tiling_on_sc=False),
        scratch_types=[
            pltpu.VMEM((_SLICE * 4,), jnp.float32),
            pltpu.VMEM((_SLICE,), jnp.float32),
            pltpu.VMEM((_SLICE,), jnp.float32),
            pltpu.VMEM((_SLICE,), jnp.float32),
            pltpu.VMEM((_SLICE,), jnp.float32),
            pltpu.VMEM((_SLICE,), jnp.float32),
            pltpu.VMEM((16,), jnp.int32),
            pltpu.VMEM((_SLICE,), jnp.float32),
            pltpu.VMEM((_OROWS * _OW,), jnp.float32),
            pltpu.VMEM((16,), jnp.float32),
            pltpu.VMEM((16, 16), jnp.float32),
            pltpu.VMEM_SHARED((2, 16, 16), jnp.float32),
        ],
    )
    out = f(bflat, s, cnt)
    return out.reshape(_N, _OW)


def kernel(boxes, scores, counts):
    return _nms_sc(boxes, scores, counts)
